# baseline (speedup 1.0000x reference)
import functools

import jax
import jax.numpy as jnp
from jax import lax
from jax.experimental import pallas as pl
from jax.experimental.pallas import tpu as pltpu
from jax.experimental.pallas import tpu_sc as plsc

N = 10000
E = 160000
D_IN = 256
D_E = 16
DH = D_IN // 2

NC = 2
NS = 16
CB = 96
E_PAD = 165888
NBN = E_PAD // NS // CB
NBA = E_PAD // (NC * NS) // CB
NROWS = N + 16

_sc_mesh = plsc.VectorSubcoreMesh(
    core_axis_name="c", subcore_axis_name="s", num_cores=NC, num_subcores=NS)


@functools.partial(
    pl.kernel,
    out_type=jax.ShapeDtypeStruct((2 * N, DH), jnp.bfloat16),
    mesh=_sc_mesh,
    scratch_types=[
        pltpu.VMEM((NBN, CB), jnp.int32),
        pltpu.VMEM((NBN, CB), jnp.int32),
        pltpu.VMEM((CB, DH), jnp.bfloat16),
        pltpu.VMEM((CB, DH), jnp.bfloat16),
        pltpu.VMEM((64, DH), jnp.bfloat16),
        pltpu.VMEM_SHARED((NROWS, DH), jnp.bfloat16),
        pltpu.VMEM_SHARED((NROWS, DH), jnp.bfloat16),
        pltpu.SemaphoreType.DMA,
        pltpu.SemaphoreType.DMA,
        pltpu.SemaphoreType.DMA,
        pltpu.SemaphoreType.DMA,
    ],
    compiler_params=pltpu.CompilerParams(use_tc_tiling_on_sc=False),
)
def _sc_nfeats(srcs_hbm, dst_hbm, nflr_hbm, aggn_hbm,
               gidx2, didx2, rows0, rows1, zwide, acc_n, table,
               sg0, sg1, ss0, ss1):
    cid = lax.axis_index("c")
    sid = lax.axis_index("s")
    rowsl = [rows0, rows1]
    sg = [sg0, sg1]
    ss = [ss0, ss1]

    zv = jnp.zeros((32,), jnp.bfloat16)
    for r in range(64):
        for k in range(DH // 32):
            zwide[r, pl.ds(k * 32, 32)] = zv

    zbase = sid * (NROWS // NS)
    for j in range(9):
        pltpu.sync_copy(zwide, acc_n.at[pl.ds(zbase + j * 64, 64)])
    pltpu.sync_copy(zwide.at[pl.ds(0, 50)], acc_n.at[pl.ds(zbase + 576, 50)])

    pltpu.sync_copy(srcs_hbm.at[pl.ds(sid * NBN, NBN)], gidx2)
    pltpu.sync_copy(dst_hbm.at[pl.ds(sid * NBN, NBN)], didx2)

    tb = sid * (N // NS)
    pltpu.sync_copy(nflr_hbm.at[pl.ds(cid * N + tb, N // NS)],
                    table.at[pl.ds(tb, N // NS)])

    plsc.subcore_barrier()

    for b in range(2):
        pltpu.async_copy(table.at[gidx2.at[b]], rowsl[b], sg[b])

    def nstep(t, _):
        for b in range(2):
            g = t * 2 + b
            pltpu.make_async_copy(table.at[gidx2.at[0]], rowsl[b], sg[b]).wait()
            pltpu.async_copy(rowsl[b], acc_n.at[didx2.at[g]], ss[b], add=True)
        for b in range(2):
            g = t * 2 + b
            pltpu.make_async_copy(rowsl[b], acc_n.at[didx2.at[0]], ss[b]).wait()
            gn = g + 2

            @pl.when(gn < NBN)
            def _():
                pltpu.async_copy(table.at[gidx2.at[gn]], rowsl[b], sg[b])

        return 0

    lax.fori_loop(0, NBN // 2, nstep, 0)

    plsc.subcore_barrier()

    obase = sid * (N // NS)
    pltpu.sync_copy(acc_n.at[pl.ds(obase, N // NS)],
                    aggn_hbm.at[pl.ds(cid * N + obase, N // NS)])


@functools.partial(
    pl.kernel,
    out_type=jax.ShapeDtypeStruct((2 * N, 32), jnp.bfloat16),
    mesh=_sc_mesh,
    scratch_types=[
        pltpu.VMEM((NBA, CB), jnp.int32),
        pltpu.VMEM((CB, 32), jnp.bfloat16),
        pltpu.VMEM((CB, 32), jnp.bfloat16),
        pltpu.VMEM((64, 32), jnp.bfloat16),
        pltpu.VMEM_SHARED((NROWS, 32), jnp.bfloat16),
        pltpu.SemaphoreType.DMA,
        pltpu.SemaphoreType.DMA,
        pltpu.SemaphoreType.DMA,
        pltpu.SemaphoreType.DMA,
    ],
    compiler_params=pltpu.CompilerParams(use_tc_tiling_on_sc=False),
)
def _sc_aux(dst_hbm, efe2_hbm, aux_hbm,
            daux2, auxb0, auxb1, znarrow, acc_x,
            sg0, sg1, ss0, ss1):
    cid = lax.axis_index("c")
    sid = lax.axis_index("s")
    auxl = [auxb0, auxb1]
    sg = [sg0, sg1]
    ss = [ss0, ss1]

    zv = jnp.zeros((32,), jnp.bfloat16)
    for r in range(64):
        znarrow[r, pl.ds(0, 32)] = zv

    zbase = sid * (NROWS // NS)
    for j in range(9):
        pltpu.sync_copy(znarrow, acc_x.at[pl.ds(zbase + j * 64, 64)])
    pltpu.sync_copy(znarrow.at[pl.ds(0, 50)], acc_x.at[pl.ds(zbase + 576, 50)])

    pltpu.sync_copy(dst_hbm.at[pl.ds((cid * NS + sid) * NBA, NBA)], daux2)

    plsc.subcore_barrier()

    base_a = (cid * NS + sid) * NBA * CB

    for b in range(2):
        pltpu.async_copy(efe2_hbm.at[pl.ds(base_a + b * CB, CB)], auxl[b], sg[b])

    def astep(t, _):
        for b in range(2):
            g = t * 2 + b
            pltpu.make_async_copy(efe2_hbm.at[pl.ds(0, CB)], auxl[b], sg[b]).wait()
            pltpu.async_copy(auxl[b], acc_x.at[daux2.at[g]], ss[b], add=True)
        for b in range(2):
            g = t * 2 + b
            pltpu.make_async_copy(auxl[b], acc_x.at[daux2.at[0]], ss[b]).wait()
            gn = g + 2

            @pl.when(gn < NBA)
            def _():
                pltpu.async_copy(
                    efe2_hbm.at[pl.ds(base_a + gn * CB, CB)], auxl[b], sg[b])

        return 0

    lax.fori_loop(0, NBA // 2, astep, 0)

    plsc.subcore_barrier()

    obase = sid * (N // NS)
    pltpu.sync_copy(acc_x.at[pl.ds(obase, N // NS)],
                    aux_hbm.at[pl.ds(cid * N + obase, N // NS)])


BLK = 2000


def _tc_pre_body(nf_ref, wa1_ref, ba_ref, out_ref):
    out_ref[...] = jnp.dot(nf_ref[...], wa1_ref[...],
                           preferred_element_type=jnp.float32) + ba_ref[...]


def _tc_body(pre_ref, al_ref, ar_ref, x0_ref, x1_ref, wm1a_ref, wm1b_ref,
             wm2_ref, wa2_ref, bm_ref, out_ref):
    aux = (x0_ref[...] + x1_ref[...]).astype(jnp.float32)
    agge = aux[:, :D_E]
    deg = aux[:, D_E:D_E + 1]
    al = al_ref[...].astype(jnp.float32)
    ar = ar_ref[...].astype(jnp.float32)
    msg = jnp.dot(al, wm1a_ref[...], preferred_element_type=jnp.float32)
    msg = msg + jnp.dot(ar, wm1b_ref[...],
                        preferred_element_type=jnp.float32)
    msg = msg + jnp.dot(agge, wm2_ref[...], preferred_element_type=jnp.float32)
    msg = msg + deg * bm_ref[...]
    h = jnp.where(deg > 0, msg * (1.0 / jnp.maximum(deg, 1.0)), 0.0)
    acc = pre_ref[...]
    acc = acc + jnp.dot(h, wa2_ref[...], preferred_element_type=jnp.float32)
    out_ref[...] = jnp.maximum(acc, 0.0)


def _tc_pre(nfeats, W_apply, b_apply):
    wa1 = W_apply[:D_IN]
    ba = b_apply.reshape(1, -1)
    nb = N // BLK
    full = lambda a: pl.BlockSpec(a.shape, lambda i: (0,) * a.ndim)
    return pl.pallas_call(
        _tc_pre_body,
        grid=(nb,),
        in_specs=[
            pl.BlockSpec((BLK, D_IN), lambda i: (i, 0)),
            full(wa1), full(ba),
        ],
        out_specs=pl.BlockSpec((BLK, D_IN), lambda i: (i, 0)),
        out_shape=jax.ShapeDtypeStruct((N, D_IN), jnp.float32),
    )(nfeats, wa1, ba)


def _tc_apply(pre, aggn2, aux2, W_msg, b_msg, W_apply, b_apply):
    wm1a = W_msg[:DH]
    wm1b = W_msg[DH:D_IN]
    wm2 = W_msg[D_IN:]
    wa2 = W_apply[D_IN:]
    bm = b_msg.reshape(1, -1)
    nb = N // BLK
    full = lambda a: pl.BlockSpec(a.shape, lambda i: (0,) * a.ndim)
    return pl.pallas_call(
        _tc_body,
        grid=(nb,),
        in_specs=[
            pl.BlockSpec((BLK, D_IN), lambda i: (i, 0)),
            pl.BlockSpec((BLK, DH), lambda i: (i, 0)),
            pl.BlockSpec((BLK, DH), lambda i: (i + nb, 0)),
            pl.BlockSpec((BLK, 32), lambda i: (i, 0)),
            pl.BlockSpec((BLK, 32), lambda i: (i + nb, 0)),
            full(wm1a), full(wm1b), full(wm2), full(wa2), full(bm),
        ],
        out_specs=pl.BlockSpec((BLK, D_IN), lambda i: (i, 0)),
        out_shape=jax.ShapeDtypeStruct((N, D_IN), jnp.float32),
    )(pre, aggn2, aggn2, aux2, aux2, wm1a, wm1b, wm2, wa2, bm)


@jax.jit
def kernel(nfeats, edge_index, efeats, W_msg, b_msg, W_apply, b_apply):
    src = edge_index[0]
    dst = edge_index[1]
    pad = E_PAD - E
    srcs2 = jnp.concatenate(
        [src, jnp.zeros((pad,), jnp.int32)]).reshape(NS * NBN, CB)
    dst_p = jnp.concatenate(
        [dst, jnp.full((pad,), N, jnp.int32)]).reshape(NS * NBN, CB)
    nflr = jnp.concatenate(
        [nfeats[:, :DH], nfeats[:, DH:]], axis=0).astype(jnp.bfloat16)
    efe2 = jnp.concatenate(
        [efeats, jnp.ones((E, 1), jnp.float32),
         jnp.zeros((E, 32 - D_E - 1), jnp.float32)], axis=1)
    efe2 = jnp.concatenate(
        [efe2, jnp.zeros((pad, 32), jnp.float32)], axis=0).astype(jnp.bfloat16)
    aggn2 = _sc_nfeats(srcs2, dst_p, nflr)
    aux2 = _sc_aux(dst_p, efe2)
    pre = _tc_pre(nfeats, W_apply, b_apply)
    return _tc_apply(pre, aggn2, aux2, W_msg, b_msg, W_apply, b_apply)


# speedup vs baseline: 1.2197x; 1.2197x over previous
import functools

import jax
import jax.numpy as jnp
from jax import lax
from jax.experimental import pallas as pl
from jax.experimental.pallas import tpu as pltpu
from jax.experimental.pallas import tpu_sc as plsc

N = 10000
E = 160000
D_IN = 256
D_E = 16
DH = D_IN // 2

NC = 2
NS = 16
CB = 96
E_PAD = 165888
NBN = E_PAD // NS // CB
NBA = E_PAD // (NC * NS) // CB
NROWS = N + 16

_sc_mesh = plsc.VectorSubcoreMesh(
    core_axis_name="c", subcore_axis_name="s", num_cores=NC, num_subcores=NS)


@functools.partial(
    pl.kernel,
    out_type=jax.ShapeDtypeStruct((2 * N, DH), jnp.bfloat16),
    mesh=_sc_mesh,
    scratch_types=[
        pltpu.VMEM((NBN, CB), jnp.int32),
        pltpu.VMEM((NBN, CB), jnp.int32),
        pltpu.VMEM((CB, DH), jnp.bfloat16),
        pltpu.VMEM((CB, DH), jnp.bfloat16),
        pltpu.VMEM((64, DH), jnp.bfloat16),
        pltpu.VMEM_SHARED((NROWS, DH), jnp.bfloat16),
        pltpu.VMEM_SHARED((NROWS, DH), jnp.bfloat16),
        pltpu.SemaphoreType.DMA,
        pltpu.SemaphoreType.DMA,
        pltpu.SemaphoreType.DMA,
        pltpu.SemaphoreType.DMA,
    ],
    compiler_params=pltpu.CompilerParams(use_tc_tiling_on_sc=False),
)
def _sc_nfeats(srcs_hbm, dst_hbm, nflr_hbm, aggn_hbm,
               gidx2, didx2, rows0, rows1, zwide, acc_n, table,
               sg0, sg1, ss0, ss1):
    cid = lax.axis_index("c")
    sid = lax.axis_index("s")
    rowsl = [rows0, rows1]
    sg = [sg0, sg1]
    ss = [ss0, ss1]

    zv = jnp.zeros((32,), jnp.bfloat16)
    for r in range(64):
        for k in range(DH // 32):
            zwide[r, pl.ds(k * 32, 32)] = zv

    zbase = sid * (NROWS // NS)
    for j in range(9):
        pltpu.sync_copy(zwide, acc_n.at[pl.ds(zbase + j * 64, 64)])
    pltpu.sync_copy(zwide.at[pl.ds(0, 50)], acc_n.at[pl.ds(zbase + 576, 50)])

    pltpu.sync_copy(srcs_hbm.at[pl.ds(sid * NBN, NBN)], gidx2)
    pltpu.sync_copy(dst_hbm.at[pl.ds(sid * NBN, NBN)], didx2)

    tb = sid * (N // NS)
    pltpu.sync_copy(nflr_hbm.at[pl.ds(cid * N + tb, N // NS)],
                    table.at[pl.ds(tb, N // NS)])

    plsc.subcore_barrier()

    for b in range(2):
        pltpu.async_copy(table.at[gidx2.at[b]], rowsl[b], sg[b])

    def nstep(t, _):
        for b in range(2):
            g = t * 2 + b
            pltpu.make_async_copy(table.at[gidx2.at[0]], rowsl[b], sg[b]).wait()
            pltpu.async_copy(rowsl[b], acc_n.at[didx2.at[g]], ss[b], add=True)
        for b in range(2):
            g = t * 2 + b
            pltpu.make_async_copy(rowsl[b], acc_n.at[didx2.at[0]], ss[b]).wait()
            gn = g + 2

            @pl.when(gn < NBN)
            def _():
                pltpu.async_copy(table.at[gidx2.at[gn]], rowsl[b], sg[b])

        return 0

    lax.fori_loop(0, NBN // 2, nstep, 0)

    plsc.subcore_barrier()

    obase = sid * (N // NS)
    pltpu.sync_copy(acc_n.at[pl.ds(obase, N // NS)],
                    aggn_hbm.at[pl.ds(cid * N + obase, N // NS)])


@functools.partial(
    pl.kernel,
    out_type=jax.ShapeDtypeStruct((2 * N, 32), jnp.float32),
    mesh=_sc_mesh,
    scratch_types=[
        pltpu.VMEM((NBA, CB), jnp.int32),
        pltpu.VMEM((CB, 32), jnp.float32),
        pltpu.VMEM((CB, 32), jnp.float32),
        pltpu.VMEM((64, 32), jnp.float32),
        pltpu.VMEM_SHARED((NROWS, 32), jnp.float32),
        pltpu.SemaphoreType.DMA,
        pltpu.SemaphoreType.DMA,
        pltpu.SemaphoreType.DMA,
        pltpu.SemaphoreType.DMA,
    ],
    compiler_params=pltpu.CompilerParams(use_tc_tiling_on_sc=False),
)
def _sc_aux(dst_hbm, efe2_hbm, aux_hbm,
            daux2, auxb0, auxb1, znarrow, acc_x,
            sg0, sg1, ss0, ss1):
    cid = lax.axis_index("c")
    sid = lax.axis_index("s")
    auxl = [auxb0, auxb1]
    sg = [sg0, sg1]
    ss = [ss0, ss1]

    zv = jnp.zeros((16,), jnp.float32)
    for r in range(64):
        for k in range(2):
            znarrow[r, pl.ds(k * 16, 16)] = zv

    zbase = sid * (NROWS // NS)
    for j in range(9):
        pltpu.sync_copy(znarrow, acc_x.at[pl.ds(zbase + j * 64, 64)])
    pltpu.sync_copy(znarrow.at[pl.ds(0, 50)], acc_x.at[pl.ds(zbase + 576, 50)])

    pltpu.sync_copy(dst_hbm.at[pl.ds((cid * NS + sid) * NBA, NBA)], daux2)

    plsc.subcore_barrier()

    base_a = (cid * NS + sid) * NBA * CB

    for b in range(2):
        pltpu.async_copy(efe2_hbm.at[pl.ds(base_a + b * CB, CB)], auxl[b], sg[b])

    def astep(t, _):
        for b in range(2):
            g = t * 2 + b
            pltpu.make_async_copy(efe2_hbm.at[pl.ds(0, CB)], auxl[b], sg[b]).wait()
            pltpu.async_copy(auxl[b], acc_x.at[daux2.at[g]], ss[b], add=True)
        for b in range(2):
            g = t * 2 + b
            pltpu.make_async_copy(auxl[b], acc_x.at[daux2.at[0]], ss[b]).wait()
            gn = g + 2

            @pl.when(gn < NBA)
            def _():
                pltpu.async_copy(
                    efe2_hbm.at[pl.ds(base_a + gn * CB, CB)], auxl[b], sg[b])

        return 0

    lax.fori_loop(0, NBA // 2, astep, 0)

    plsc.subcore_barrier()

    obase = sid * (N // NS)
    pltpu.sync_copy(acc_x.at[pl.ds(obase, N // NS)],
                    aux_hbm.at[pl.ds(cid * N + obase, N // NS)])


BLK = 2000


def _tc_pre_body(nf_ref, wa1_ref, ba_ref, out_ref):
    nf = nf_ref[...].astype(jnp.bfloat16)
    out_ref[...] = jnp.dot(nf, wa1_ref[...],
                           preferred_element_type=jnp.float32) + ba_ref[...]


def _tc_body(pre_ref, al_ref, ar_ref, x0_ref, x1_ref, wm1a_ref, wm1b_ref,
             wm2_ref, wa2_ref, bm_ref, out_ref):
    aux = x0_ref[...] + x1_ref[...]
    agge = aux[:, :D_E]
    deg = aux[:, D_E:D_E + 1]
    msg = jnp.dot(al_ref[...], wm1a_ref[...],
                  preferred_element_type=jnp.float32)
    msg = msg + jnp.dot(ar_ref[...], wm1b_ref[...],
                        preferred_element_type=jnp.float32)
    msg = msg + jnp.dot(agge, wm2_ref[...], preferred_element_type=jnp.float32)
    msg = msg + deg * bm_ref[...]
    h = jnp.where(deg > 0, msg * (1.0 / jnp.maximum(deg, 1.0)), 0.0)
    acc = pre_ref[...]
    acc = acc + jnp.dot(h.astype(jnp.bfloat16), wa2_ref[...],
                        preferred_element_type=jnp.float32)
    out_ref[...] = jnp.maximum(acc, 0.0)


def _tc_pre(nfeats, W_apply, b_apply):
    wa1 = W_apply[:D_IN].astype(jnp.bfloat16)
    ba = b_apply.reshape(1, -1)
    nb = N // BLK
    full = lambda a: pl.BlockSpec(a.shape, lambda i: (0,) * a.ndim)
    return pl.pallas_call(
        _tc_pre_body,
        grid=(nb,),
        in_specs=[
            pl.BlockSpec((BLK, D_IN), lambda i: (i, 0)),
            full(wa1), full(ba),
        ],
        out_specs=pl.BlockSpec((BLK, D_IN), lambda i: (i, 0)),
        out_shape=jax.ShapeDtypeStruct((N, D_IN), jnp.float32),
    )(nfeats, wa1, ba)


def _tc_apply(pre, aggn2, aux2, W_msg, b_msg, W_apply, b_apply):
    wm1a = W_msg[:DH].astype(jnp.bfloat16)
    wm1b = W_msg[DH:D_IN].astype(jnp.bfloat16)
    wm2 = W_msg[D_IN:]
    wa2 = W_apply[D_IN:].astype(jnp.bfloat16)
    bm = b_msg.reshape(1, -1)
    nb = N // BLK
    full = lambda a: pl.BlockSpec(a.shape, lambda i: (0,) * a.ndim)
    return pl.pallas_call(
        _tc_body,
        grid=(nb,),
        in_specs=[
            pl.BlockSpec((BLK, D_IN), lambda i: (i, 0)),
            pl.BlockSpec((BLK, DH), lambda i: (i, 0)),
            pl.BlockSpec((BLK, DH), lambda i: (i + nb, 0)),
            pl.BlockSpec((BLK, 32), lambda i: (i, 0)),
            pl.BlockSpec((BLK, 32), lambda i: (i + nb, 0)),
            full(wm1a), full(wm1b), full(wm2), full(wa2), full(bm),
        ],
        out_specs=pl.BlockSpec((BLK, D_IN), lambda i: (i, 0)),
        out_shape=jax.ShapeDtypeStruct((N, D_IN), jnp.float32),
    )(pre, aggn2, aggn2, aux2, aux2, wm1a, wm1b, wm2, wa2, bm)


@jax.jit
def kernel(nfeats, edge_index, efeats, W_msg, b_msg, W_apply, b_apply):
    src = edge_index[0]
    dst = edge_index[1]
    pad = E_PAD - E
    srcs2 = jnp.concatenate(
        [src, jnp.zeros((pad,), jnp.int32)]).reshape(NS * NBN, CB)
    dst_p = jnp.concatenate(
        [dst, jnp.full((pad,), N, jnp.int32)]).reshape(NS * NBN, CB)
    nflr = jnp.concatenate(
        [nfeats[:, :DH], nfeats[:, DH:]], axis=0).astype(jnp.bfloat16)
    efe2 = jnp.concatenate(
        [efeats, jnp.ones((E, 1), jnp.float32),
         jnp.zeros((E, 32 - D_E - 1), jnp.float32)], axis=1)
    efe2 = jnp.concatenate([efe2, jnp.zeros((pad, 32), jnp.float32)], axis=0)
    aggn2 = _sc_nfeats(srcs2, dst_p, nflr)
    aux2 = _sc_aux(dst_p, efe2)
    pre = _tc_pre(nfeats, W_apply, b_apply)
    return _tc_apply(pre, aggn2, aux2, W_msg, b_msg, W_apply, b_apply)
